# baseline (speedup 1.0000x reference)
import functools

import jax
import jax.numpy as jnp
from jax import lax
from jax.experimental import pallas as pl
from jax.experimental.pallas import tpu as pltpu
from jax.experimental.pallas import tpu_sc as plsc

B = 16384
K = 5
D = 64
CB = 64
NC, NS = 2, 16
NW = NC * NS
BPW = B // NW
NCH = BPW // CB
NG = CB // 16

_mesh = plsc.VectorSubcoreMesh(core_axis_name="c", subcore_axis_name="s")


@functools.partial(
    pl.kernel,
    out_type=jax.ShapeDtypeStruct((B * K,), jnp.float32),
    mesh=_mesh,
    scratch_types=[
        pltpu.VMEM((2 * (1 + K) * 2 * CB,), jnp.int32),
        pltpu.VMEM((2 * (1 + K) * 128,), jnp.int32),
        pltpu.VMEM((2, CB, 2 * D), jnp.float32),
        pltpu.VMEM((2, K, CB, 2 * D), jnp.float32),
        pltpu.VMEM((BPW * K,), jnp.float32),
        pltpu.SemaphoreType.DMA,
        pltpu.SemaphoreType.DMA,
    ],
    compiler_params=pltpu.CompilerParams(needs_layout_passes=False,
                                         use_tc_tiling_on_sc=True),
)
def _sc_skipgram(tgt_hbm, ctxt_hbm, ttab_hbm, otab_hbm, out_hbm,
                 vid_v, lid_v, trows_v, crows_v, outb_v,
                 sem0, sem1):
    wid = lax.axis_index("s") * NC + lax.axis_index("c")
    b0w = wid * BPW
    sems = [sem0, sem1]
    descs = [None, None]

    VSLOT = (1 + K) * 2 * CB

    def stage_pair(p):
        q = (p & 1) * VSLOT
        pltpu.sync_copy(tgt_hbm.at[pl.ds(b0w + p * 2 * CB, 2 * CB)],
                        vid_v.at[pl.ds(q, 2 * CB)])
        for k in range(K):
            pltpu.sync_copy(
                ctxt_hbm.at[pl.ds(k * B + b0w + p * 2 * CB, 2 * CB)],
                vid_v.at[pl.ds(q + (1 + k) * 2 * CB, 2 * CB)])

    def fire(c):
        s = c % 2
        q = ((c // 2) & 1) * VSLOT
        for r in range(1 + K):
            for g in range(NG):
                sl = pl.ds(q + r * 2 * CB + (c & 1) * CB + g * 16, 16)
                dl = pl.ds((s * (1 + K) + r) * 128 + g * 16, 16)
                lid_v[dl] = lax.shift_right_logical(vid_v[sl], 1)
        ds = [pltpu.async_copy(
            ttab_hbm.at[lid_v.at[pl.ds(s * (1 + K) * 128, CB)]],
            trows_v.at[s], sems[s])]
        for k in range(K):
            ds.append(pltpu.async_copy(
                otab_hbm.at[lid_v.at[pl.ds((s * (1 + K) + 1 + k) * 128, CB)]],
                crows_v.at[s, k], sems[s]))
        descs[s] = ds

    def compute(c):
        s = c % 2
        trows = trows_v.at[s]
        crows = crows_v.at[s]
        q = ((c // 2) & 1) * VSLOT
        for g in range(NG):
            rowg = lax.iota(jnp.int32, 16) + g * 16
            sl = lambda r: pl.ds(q + r * 2 * CB + (c & 1) * CB + g * 16, 16)
            tpar = (vid_v[sl(0)] & 1) * D
            cpar = [(vid_v[sl(1 + k)] & 1) * D for k in range(K)]

            def body(d, accs):
                tcol = plsc.load_gather(trows, [rowg, tpar + d])
                return tuple(
                    accs[k] + tcol * plsc.load_gather(crows.at[k],
                                                      [rowg, cpar[k] + d])
                    for k in range(K))

            accs = lax.fori_loop(
                0, D, body,
                tuple(jnp.zeros((16,), jnp.float32) for _ in range(K)))
            pbase = (rowg + c * CB) * K
            for k in range(K):
                plsc.store_scatter(outb_v, [pbase + k], accs[k])

    stage_pair(0)
    fire(0)
    for c in range(NCH):
        if c + 1 < NCH:
            if (c + 1) % 2 == 0:
                stage_pair((c + 1) // 2)
            fire(c + 1)
        for d in descs[c % 2]:
            d.wait()
        compute(c)
    pltpu.sync_copy(outb_v, out_hbm.at[pl.ds(wid * BPW * K, BPW * K)])


def kernel(target, context, target_table, output_table):
    ctx_t = context.astype(jnp.int32).T.reshape(-1)
    tgt1 = target.astype(jnp.int32)
    ttab2 = target_table.reshape(-1, 2 * D)
    otab2 = output_table.reshape(-1, 2 * D)
    out2 = _sc_skipgram(tgt1, ctx_t, ttab2, otab2)
    return out2.reshape(B, K)


# speedup vs baseline: 1.0596x; 1.0596x over previous
import functools

import jax
import jax.numpy as jnp
from jax import lax
from jax.experimental import pallas as pl
from jax.experimental.pallas import tpu as pltpu
from jax.experimental.pallas import tpu_sc as plsc

B = 16384
K = 5
D = 64
CB = 64
NC, NS = 2, 16
NW = NC * NS
BPW = B // NW
NCH = BPW // CB
NG = CB // 16

_mesh = plsc.VectorSubcoreMesh(core_axis_name="c", subcore_axis_name="s")


@functools.partial(
    pl.kernel,
    out_type=jax.ShapeDtypeStruct((B * K,), jnp.float32),
    mesh=_mesh,
    scratch_types=[
        pltpu.VMEM((2 * (1 + K) * 2 * CB,), jnp.int32),
        pltpu.VMEM((2, CB, 2 * D), jnp.float32),
        pltpu.VMEM((2, K, CB, 2 * D), jnp.float32),
        pltpu.VMEM((BPW * K,), jnp.float32),
        pltpu.SemaphoreType.DMA,
        pltpu.SemaphoreType.DMA,
    ],
    compiler_params=pltpu.CompilerParams(needs_layout_passes=False,
                                         use_tc_tiling_on_sc=True),
)
def _sc_skipgram(tgt_hbm, ctxt_hbm, ttab_hbm, otab_hbm, out_hbm,
                 vid_v, trows_v, crows_v, outb_v,
                 sem0, sem1):
    wid = lax.axis_index("s") * NC + lax.axis_index("c")
    b0w = wid * BPW
    sems = [sem0, sem1]
    descs = [None, None]

    VSLOT = (1 + K) * 2 * CB

    def stage_pair(p):
        q = (p & 1) * VSLOT
        pltpu.sync_copy(tgt_hbm.at[pl.ds(b0w + p * 2 * CB, 2 * CB)],
                        vid_v.at[pl.ds(q, 2 * CB)])
        for k in range(K):
            pltpu.sync_copy(
                ctxt_hbm.at[pl.ds(k * B + b0w + p * 2 * CB, 2 * CB)],
                vid_v.at[pl.ds(q + (1 + k) * 2 * CB, 2 * CB)])

    def idslice(c, r):
        q = ((c // 2) & 1) * VSLOT
        return pl.ds(q + r * 2 * CB + (c & 1) * CB, CB)

    def fire(c):
        s = c % 2
        ds = [pltpu.async_copy(ttab_hbm.at[vid_v.at[idslice(c, 0)]],
                               trows_v.at[s], sems[s])]
        for k in range(K):
            ds.append(pltpu.async_copy(otab_hbm.at[vid_v.at[idslice(c, 1 + k)]],
                                       crows_v.at[s, k], sems[s]))
        descs[s] = ds

    def compute(c):
        s = c % 2
        trows = trows_v.at[s]
        crows = crows_v.at[s]
        for g in range(NG):
            rowg = lax.iota(jnp.int32, 16) + g * 16

            def body(d, accs):
                dsplat = jnp.zeros((16,), jnp.int32) + d
                tcol = plsc.load_gather(trows, [rowg, dsplat])
                return tuple(
                    accs[k] + tcol * plsc.load_gather(crows.at[k],
                                                      [rowg, dsplat])
                    for k in range(K))

            accs = lax.fori_loop(
                0, D, body,
                tuple(jnp.zeros((16,), jnp.float32) for _ in range(K)))
            pbase = (rowg + c * CB) * K
            for k in range(K):
                plsc.store_scatter(outb_v, [pbase + k], accs[k])

    stage_pair(0)
    fire(0)
    for c in range(NCH):
        if c + 1 < NCH:
            if (c + 1) % 2 == 0:
                stage_pair((c + 1) // 2)
            fire(c + 1)
        for d in descs[c % 2]:
            d.wait()
        compute(c)
    pltpu.sync_copy(outb_v, out_hbm.at[pl.ds(wid * BPW * K, BPW * K)])


def kernel(target, context, target_table, output_table):
    ctx_t = context.astype(jnp.int32).T.reshape(-1)
    tgt1 = target.astype(jnp.int32)
    ttab_p = jnp.pad(target_table, ((0, 0), (0, D)))
    otab_p = jnp.pad(output_table, ((0, 0), (0, D)))
    out2 = _sc_skipgram(tgt1, ctx_t, ttab_p, otab_p)
    return out2.reshape(B, K)
